# baseline (speedup 1.0000x reference)
import jax
import jax.numpy as jnp
from jax.experimental import pallas as pl

_H = 256
_D = 8
_OFF = [0, 524, 1048, 1572]
_IN_ROWS = [_D, _D, _D, 6 * _D]
_ROWS = 2136


def _dot(a, b):
    return jnp.dot(a, b, preferred_element_type=jnp.float32)


def _slices(p_ref, f):
    o, r = _OFF[f], _IN_ROWS[f]
    w1 = p_ref[o:o + r, :]
    b1 = p_ref[o + r:o + r + 1, :]
    w2 = p_ref[o + r + 1:o + r + 257, :]
    b2 = p_ref[o + r + 257:o + r + 258, :]
    w3 = p_ref[o + r + 258:o + r + 514, :]
    b3 = p_ref[o + r + 514:o + r + 515, :]
    b4 = p_ref[o + r + 515:o + r + 516, 0:_D]
    return w1, b1, w2, b2, w3, b3, b4


def _fused_kernel(x_ref, mm_ref, p_ref, w14, w24, w34, w44, out_ref):
    xc = x_ref[0]
    init = xc
    mm = mm_ref[0]
    mmc = mm[:, None]
    mmr = mm[None, :]
    prev = (mmr == mmc - 1).astype(jnp.float32)
    nxt = (mmr == mmc + 1).astype(jnp.float32)

    w11, b11, w12, b12, w13, b13, b14 = _slices(p_ref, 0)
    w21, b21, w22, b22, w23, b23, b24 = _slices(p_ref, 1)
    _, b31, w32, b32, w33, b33, b34 = _slices(p_ref, 2)
    w41, b41, w42, b42, w43, b43, b44 = _slices(p_ref, 3)

    h3 = jax.nn.relu(b31)
    h3 = jax.nn.relu(_dot(h3, w32) + b32)
    h3 = jax.nn.relu(_dot(h3, w33) + b33)
    a3 = jax.nn.relu(_dot(h3, w34[...]) + b34)

    c_const = _dot(a3, w41[16:24, :]) + _dot(init, w41[40:48, :]) + b41

    for _ in range(3):
        in1 = _dot(prev, xc)
        in2 = _dot(nxt, xc)

        h1 = jax.nn.relu(_dot(in1, w11) + b11)
        h2 = jax.nn.relu(_dot(in2, w21) + b21)
        h1 = jax.nn.relu(_dot(h1, w12) + b12)
        h2 = jax.nn.relu(_dot(h2, w22) + b22)
        h1 = jax.nn.relu(_dot(h1, w13) + b13)
        h2 = jax.nn.relu(_dot(h2, w23) + b23)
        a1 = jax.nn.relu(_dot(h1, w14[...]) + b14)
        a2 = jax.nn.relu(_dot(h2, w24[...]) + b24)

        a4 = jax.nn.relu(jnp.sum(xc, axis=0, keepdims=True))

        h = (_dot(a1, w41[0:8, :]) + _dot(a2, w41[8:16, :])
             + _dot(a4, w41[24:32, :]) + _dot(xc, w41[32:40, :]) + c_const)
        h = jax.nn.relu(h)
        h = jax.nn.relu(_dot(h, w42) + b42)
        h = jax.nn.relu(_dot(h, w43) + b43)
        xc = _dot(h, w44[...]) + b44

    out_ref[0] = xc


def _pack(params):
    pieces = []
    w4s = []
    for name in ("f1", "f2", "f3", "f4"):
        (w1, b1), (w2, b2), (w3, b3), (w4, b4) = params[name]
        pieces += [w1, b1[None, :], w2, b2[None, :], w3, b3[None, :],
                   jnp.pad(b4, (0, _H - _D))[None, :]]
        w4s.append(w4)
    return jnp.concatenate(pieces, axis=0), w4s


def kernel(x, MM, PM, params):
    J, N, d = x.shape
    packed, w4s = _pack(params)
    out = pl.pallas_call(
        _fused_kernel,
        out_shape=jax.ShapeDtypeStruct((J, N, d), jnp.float32),
    )(x, MM, packed, *w4s)
    return out


# speedup vs baseline: 4.0710x; 4.0710x over previous
import jax
import jax.numpy as jnp
from jax.experimental import pallas as pl


def _dot(a, b):
    return jnp.dot(a, b, preferred_element_type=jnp.float32)


def _fused_kernel(x_ref, mm_ref,
                  w11, b11, w12, b12, w13, b13, w14, b14,
                  w21, b21, w22, b22, w23, b23, w24, b24,
                  b31, w32, b32, w33, b33, w34, b34,
                  w41, b41, w42, b42, w43, b43, w44, b44,
                  out_ref):
    xc = x_ref[0]
    init = xc
    mm = mm_ref[0]
    mmc = mm[:, None]
    mmr = mm[None, :]
    prev = (mmr == mmc - 1).astype(jnp.float32)
    nxt = (mmr == mmc + 1).astype(jnp.float32)

    h3 = jax.nn.relu(b31[...][None, :])
    h3 = jax.nn.relu(_dot(h3, w32[...]) + b32[...])
    h3 = jax.nn.relu(_dot(h3, w33[...]) + b33[...])
    a3 = jax.nn.relu(_dot(h3, w34[...]) + b34[...])

    c_const = (_dot(a3, w41[16:24, :]) + _dot(init, w41[40:48, :])
               + b41[...][None, :])

    for _ in range(3):
        in1 = _dot(prev, xc)
        in2 = _dot(nxt, xc)

        h1 = jax.nn.relu(_dot(in1, w11[...]) + b11[...])
        h2 = jax.nn.relu(_dot(in2, w21[...]) + b21[...])
        h1 = jax.nn.relu(_dot(h1, w12[...]) + b12[...])
        h2 = jax.nn.relu(_dot(h2, w22[...]) + b22[...])
        h1 = jax.nn.relu(_dot(h1, w13[...]) + b13[...])
        h2 = jax.nn.relu(_dot(h2, w23[...]) + b23[...])
        a1 = jax.nn.relu(_dot(h1, w14[...]) + b14[...])
        a2 = jax.nn.relu(_dot(h2, w24[...]) + b24[...])

        a4 = jax.nn.relu(jnp.sum(xc, axis=0, keepdims=True))

        h = (_dot(a1, w41[0:8, :]) + _dot(a2, w41[8:16, :])
             + _dot(a4, w41[24:32, :]) + _dot(xc, w41[32:40, :]) + c_const)
        h = jax.nn.relu(h)
        h = jax.nn.relu(_dot(h, w42[...]) + b42[...])
        h = jax.nn.relu(_dot(h, w43[...]) + b43[...])
        xc = _dot(h, w44[...]) + b44[...]

    out_ref[0] = xc


def kernel(x, MM, PM, params):
    J, N, d = x.shape
    (f1w1, f1b1), (f1w2, f1b2), (f1w3, f1b3), (f1w4, f1b4) = params["f1"]
    (f2w1, f2b1), (f2w2, f2b2), (f2w3, f2b3), (f2w4, f2b4) = params["f2"]
    (_unused_w31, f3b1), (f3w2, f3b2), (f3w3, f3b3), (f3w4, f3b4) = params["f3"]
    (f4w1, f4b1), (f4w2, f4b2), (f4w3, f4b3), (f4w4, f4b4) = params["f4"]
    flat = [f1w1, f1b1, f1w2, f1b2, f1w3, f1b3, f1w4, f1b4,
            f2w1, f2b1, f2w2, f2b2, f2w3, f2b3, f2w4, f2b4,
            f3b1, f3w2, f3b2, f3w3, f3b3, f3w4, f3b4,
            f4w1, f4b1, f4w2, f4b2, f4w3, f4b3, f4w4, f4b4]
    out = pl.pallas_call(
        _fused_kernel,
        out_shape=jax.ShapeDtypeStruct((J, N, d), jnp.float32),
    )(x, MM, *flat)
    return out


# speedup vs baseline: 4.0891x; 1.0044x over previous
import jax
import jax.numpy as jnp
from jax.experimental import pallas as pl


def _dot(a, b):
    return jnp.dot(a, b, preferred_element_type=jnp.float32)


def _fused_kernel(x_ref, mm_ref,
                  w11, b11, w12, b12, w13, b13, w14, b14,
                  w21, b21, w22, b22, w23, b23, w24, b24,
                  b31, w32, b32, w33, b33, w34, b34,
                  w41, b41, w42, b42, w43, b43, w44, b44,
                  out_ref):
    xc = x_ref[0]
    init = xc

    h3 = jax.nn.relu(b31[...][None, :])
    h3 = jax.nn.relu(_dot(h3, w32[...]) + b32[...])
    h3 = jax.nn.relu(_dot(h3, w33[...]) + b33[...])
    a3 = jax.nn.relu(_dot(h3, w34[...]) + b34[...])

    c_const = (_dot(a3, w41[16:24, :]) + _dot(init, w41[40:48, :])
               + b41[...][None, :])

    mm = mm_ref[0]
    mmc = mm[:, None]
    mmr = mm[None, :]
    prev = (mmr == mmc - 1).astype(jnp.float32)
    nxt = (mmr == mmc + 1).astype(jnp.float32)

    for _ in range(3):
        in1 = _dot(prev, xc)
        in2 = _dot(nxt, xc)

        h1 = jax.nn.relu(_dot(in1, w11[...]) + b11[...])
        h2 = jax.nn.relu(_dot(in2, w21[...]) + b21[...])
        h1 = jax.nn.relu(_dot(h1, w12[...]) + b12[...])
        h2 = jax.nn.relu(_dot(h2, w22[...]) + b22[...])
        h1 = jax.nn.relu(_dot(h1, w13[...]) + b13[...])
        h2 = jax.nn.relu(_dot(h2, w23[...]) + b23[...])
        a1 = jax.nn.relu(_dot(h1, w14[...]) + b14[...])
        a2 = jax.nn.relu(_dot(h2, w24[...]) + b24[...])

        a4 = jax.nn.relu(jnp.sum(xc, axis=0, keepdims=True))

        h = (_dot(a1, w41[0:8, :]) + _dot(a2, w41[8:16, :])
             + _dot(a4, w41[24:32, :]) + _dot(xc, w41[32:40, :]) + c_const)
        h = jax.nn.relu(h)
        h = jax.nn.relu(_dot(h, w42[...]) + b42[...])
        h = jax.nn.relu(_dot(h, w43[...]) + b43[...])
        xc = _dot(h, w44[...]) + b44[...]

    out_ref[0] = xc


def kernel(x, MM, PM, params):
    J, N, d = x.shape
    (f1w1, f1b1), (f1w2, f1b2), (f1w3, f1b3), (f1w4, f1b4) = params["f1"]
    (f2w1, f2b1), (f2w2, f2b2), (f2w3, f2b3), (f2w4, f2b4) = params["f2"]
    (_unused_w31, f3b1), (f3w2, f3b2), (f3w3, f3b3), (f3w4, f3b4) = params["f3"]
    (f4w1, f4b1), (f4w2, f4b2), (f4w3, f4b3), (f4w4, f4b4) = params["f4"]
    flat = [f1w1, f1b1, f1w2, f1b2, f1w3, f1b3, f1w4, f1b4,
            f2w1, f2b1, f2w2, f2b2, f2w3, f2b3, f2w4, f2b4,
            f3b1, f3w2, f3b2, f3w3, f3b3, f3w4, f3b4,
            f4w1, f4b1, f4w2, f4b2, f4w3, f4b3, f4w4, f4b4]
    out = pl.pallas_call(
        _fused_kernel,
        out_shape=jax.ShapeDtypeStruct((J, N, d), jnp.float32),
    )(x, MM, *flat)
    return out
